# baseline (speedup 1.0000x reference)
import functools

import jax
import jax.numpy as jnp
from jax import lax
from jax.experimental import pallas as pl
from jax.experimental.pallas import tpu as pltpu
from jax.experimental.pallas import tpu_sc as plsc

N_NUC = 10000
N_CENTER = 320000
MAX_CHARGE = 100
FEAT = 128

NC, NS = 2, 16
NW = NC * NS
PER_W = N_CENTER // NW
CHUNK = 200
NGROUP = 10
MPG = 4
NCHUNK_M = NGROUP * MPG
NCHUNK_A = NGROUP
MAIN_SPAN = NCHUNK_M * CHUNK

_mesh = plsc.VectorSubcoreMesh(core_axis_name="c", subcore_axis_name="s")


@functools.partial(
    pl.kernel,
    out_type=jax.ShapeDtypeStruct((N_CENTER, FEAT), jnp.float32),
    mesh=_mesh,
    scratch_types=(
        [pltpu.VMEM_SHARED((N_NUC,), jnp.int32)]
        + [pltpu.VMEM_SHARED((MAX_CHARGE, FEAT), jnp.float32)]
        + [pltpu.VMEM((CHUNK,), jnp.int32) for _ in range(4)]
        + [pltpu.VMEM((CHUNK,), jnp.int32) for _ in range(4)]
        + [pltpu.VMEM((CHUNK, FEAT), jnp.float32) for _ in range(4)]
        + [pltpu.SemaphoreType.DMA for _ in range(16)]
    ),
)
def _two_level_gather(table_hbm, charges_hbm, cidx_hbm, out_hbm, *refs):
    fc_s = refs[0]
    tab_s = refs[1]
    cidx_m, cidx_a = refs[2:4], refs[4:6]
    chg_m, chg_a = refs[6:8], refs[8:10]
    rows_m, rows_a = refs[10:12], refs[12:14]
    sems = refs[14:]
    sem_im, sem_ia = sems[0:2], sems[2:4]
    sem_cm, sem_ca = sems[4:6], sems[6:8]
    sem_rm, sem_ra = sems[8:10], sems[10:12]
    sem_sm, sem_sa = sems[12:14], sems[14:16]

    wid = lax.axis_index("s") * NC + lax.axis_index("c")
    base = wid * PER_W
    base_a = base + MAIN_SPAN

    pltpu.async_copy(cidx_hbm.at[pl.ds(base, CHUNK)], cidx_m[0], sem_im[0])
    pltpu.async_copy(cidx_hbm.at[pl.ds(base + CHUNK, CHUNK)], cidx_m[1], sem_im[1])
    pltpu.async_copy(cidx_hbm.at[pl.ds(base_a, CHUNK)], cidx_a[0], sem_ia[0])
    pltpu.async_copy(cidx_hbm.at[pl.ds(base_a + CHUNK, CHUNK)], cidx_a[1], sem_ia[1])

    @pl.when(lax.axis_index("s") == 0)
    def _():
        pltpu.sync_copy(charges_hbm, fc_s)

    @pl.when(lax.axis_index("s") == 1)
    def _():
        pltpu.sync_copy(table_hbm, tab_s)
    plsc.subcore_barrier()

    def fire_cidx_m(c, b):
        pltpu.async_copy(
            cidx_hbm.at[pl.ds(base + c * CHUNK, CHUNK)], cidx_m[b], sem_im[b])

    def wait_cidx_m(b):
        pltpu.make_async_copy(
            cidx_hbm.at[pl.ds(0, CHUNK)], cidx_m[b], sem_im[b]).wait()

    def fire_chg_m(b):
        pltpu.async_copy(fc_s.at[cidx_m[b]], chg_m[b], sem_cm[b])

    def wait_chg_m(b):
        pltpu.make_async_copy(fc_s.at[cidx_m[b]], chg_m[b], sem_cm[b]).wait()

    def fire_rows_m(b):
        pltpu.async_copy(tab_s.at[chg_m[b]], rows_m[b], sem_rm[b])

    def wait_rows_m(b):
        pltpu.make_async_copy(tab_s.at[chg_m[b]], rows_m[b], sem_rm[b]).wait()

    def fire_store_m(c, b):
        pltpu.async_copy(
            rows_m[b], out_hbm.at[pl.ds(base + c * CHUNK, CHUNK)], sem_sm[b])

    def wait_store_m(b):
        pltpu.make_async_copy(
            rows_m[b], out_hbm.at[pl.ds(0, CHUNK)], sem_sm[b]).wait()

    def fire_cidx_a(g, a):
        pltpu.async_copy(
            cidx_hbm.at[pl.ds(base_a + g * CHUNK, CHUNK)], cidx_a[a], sem_ia[a])

    def wait_cidx_a(a):
        pltpu.make_async_copy(
            cidx_hbm.at[pl.ds(0, CHUNK)], cidx_a[a], sem_ia[a]).wait()

    def fire_chg_a(a):
        pltpu.async_copy(fc_s.at[cidx_a[a]], chg_a[a], sem_ca[a])

    def wait_chg_a(a):
        pltpu.make_async_copy(fc_s.at[cidx_a[a]], chg_a[a], sem_ca[a]).wait()

    def fire_rows_a(a):
        pltpu.async_copy(table_hbm.at[chg_a[a]], rows_a[a], sem_ra[a])

    def wait_rows_a(a):
        pltpu.make_async_copy(table_hbm.at[chg_a[a]], rows_a[a], sem_ra[a]).wait()

    def fire_store_a(g, a):
        pltpu.async_copy(
            rows_a[a], out_hbm.at[pl.ds(base_a + g * CHUNK, CHUNK)], sem_sa[a])

    def wait_store_a(a):
        pltpu.make_async_copy(
            rows_a[a], out_hbm.at[pl.ds(0, CHUNK)], sem_sa[a]).wait()

    wait_cidx_m(0)
    fire_chg_m(0)
    wait_cidx_a(0)
    fire_chg_a(0)

    def main_unit(c, b):
        @pl.when(c >= 2)
        def _():
            wait_store_m(b)
        wait_chg_m(b)
        fire_rows_m(b)

        @pl.when(c + 2 < NCHUNK_M)
        def _():
            fire_cidx_m(c + 2, b)

        @pl.when(c + 1 < NCHUNK_M)
        def _():
            wait_cidx_m(1 - b)
            fire_chg_m(1 - b)

        @pl.when(c >= 1)
        def _():
            wait_rows_m(1 - b)
            fire_store_m(c - 1, 1 - b)

    @pl.loop(0, NGROUP // 2)
    def _(h):
        for ga in range(2):
            g = h * 2 + ga

            @pl.when(g >= 2)
            def _():
                wait_store_a(ga)
            wait_chg_a(ga)
            fire_rows_a(ga)

            @pl.when(g + 2 < NCHUNK_A)
            def _():
                fire_cidx_a(g + 2, ga)

            for k in range(MPG):
                main_unit(g * MPG + k, k % 2)

            @pl.when(g + 1 < NCHUNK_A)
            def _():
                wait_cidx_a(1 - ga)
                fire_chg_a(1 - ga)
            wait_rows_a(ga)
            fire_store_a(g, ga)

    wait_rows_m(1)
    fire_store_m(NCHUNK_M - 1, 1)
    wait_store_m(0)
    wait_store_m(1)
    wait_store_a(0)
    wait_store_a(1)


def kernel(table, flat_charges, center_idx):
    return _two_level_gather(
        table,
        flat_charges.astype(jnp.int32),
        center_idx.astype(jnp.int32),
    )


# speedup vs baseline: 1.9100x; 1.9100x over previous
import functools

import jax
import jax.numpy as jnp
from jax import lax
from jax.experimental import pallas as pl
from jax.experimental.pallas import tpu as pltpu
from jax.experimental.pallas import tpu_sc as plsc

N_NUC = 10000
N_CENTER = 320000
MAX_CHARGE = 100
FEAT = 128

NC, NS = 2, 16
NW = NC * NS
PER_W = N_CENTER // NW
CHUNK = 200
NCHUNK = PER_W // CHUNK
NB = 4
NRING = (NCHUNK + NB - 1) // NB

_mesh = plsc.VectorSubcoreMesh(core_axis_name="c", subcore_axis_name="s")


@functools.partial(
    pl.kernel,
    out_type=jax.ShapeDtypeStruct((N_CENTER, FEAT), jnp.float32),
    mesh=_mesh,
    scratch_types=(
        [pltpu.VMEM_SHARED((N_NUC,), jnp.int32)]
        + [pltpu.VMEM_SHARED((MAX_CHARGE, FEAT), jnp.float32)]
        + [pltpu.VMEM((CHUNK,), jnp.int32) for _ in range(NB)]
        + [pltpu.VMEM((CHUNK,), jnp.int32) for _ in range(NB)]
        + [pltpu.VMEM((CHUNK, FEAT), jnp.float32) for _ in range(NB)]
        + [pltpu.SemaphoreType.DMA for _ in range(4 * NB)]
    ),
)
def _two_level_gather(table_hbm, charges_hbm, cidx_hbm, out_hbm, *refs):
    fc_s = refs[0]
    tab_s = refs[1]
    cidx_v = refs[2:2 + NB]
    chg_v = refs[2 + NB:2 + 2 * NB]
    rows_v = refs[2 + 2 * NB:2 + 3 * NB]
    sems = refs[2 + 3 * NB:]
    sem_i = sems[0:NB]
    sem_c = sems[NB:2 * NB]
    sem_r = sems[2 * NB:3 * NB]
    sem_s = sems[3 * NB:4 * NB]

    wid = lax.axis_index("s") * NC + lax.axis_index("c")
    base = wid * PER_W

    pltpu.async_copy(
        cidx_hbm.at[pl.ds(base, CHUNK)], cidx_v[0], sem_i[0])
    pltpu.async_copy(
        cidx_hbm.at[pl.ds(base + CHUNK, CHUNK)], cidx_v[1], sem_i[1])

    @pl.when(lax.axis_index("s") == 0)
    def _():
        pltpu.sync_copy(charges_hbm, fc_s)

    @pl.when(lax.axis_index("s") == 1)
    def _():
        pltpu.sync_copy(table_hbm, tab_s)
    plsc.subcore_barrier()

    def fire_cidx(c, b):
        pltpu.async_copy(
            cidx_hbm.at[pl.ds(base + c * CHUNK, CHUNK)], cidx_v[b], sem_i[b])

    def wait_cidx(b):
        pltpu.make_async_copy(
            cidx_hbm.at[pl.ds(0, CHUNK)], cidx_v[b], sem_i[b]).wait()

    def fire_chg(b):
        pltpu.async_copy(fc_s.at[cidx_v[b]], chg_v[b], sem_c[b])

    def wait_chg(b):
        pltpu.make_async_copy(fc_s.at[cidx_v[b]], chg_v[b], sem_c[b]).wait()

    def fire_rows(b):
        pltpu.async_copy(tab_s.at[chg_v[b]], rows_v[b], sem_r[b])

    def wait_rows(b):
        pltpu.make_async_copy(tab_s.at[chg_v[b]], rows_v[b], sem_r[b]).wait()

    def fire_store(c, b):
        pltpu.async_copy(
            rows_v[b], out_hbm.at[pl.ds(base + c * CHUNK, CHUNK)], sem_s[b])

    def wait_store(b):
        pltpu.make_async_copy(
            rows_v[b], out_hbm.at[pl.ds(0, CHUNK)], sem_s[b]).wait()

    wait_cidx(0)
    fire_chg(0)

    @pl.loop(0, NRING)
    def _(g):
        for b in range(NB):
            c = g * NB + b

            @pl.when(c < NCHUNK)
            def _():
                @pl.when(c >= 1)
                def _():
                    pb = (b - 1) % NB
                    wait_rows(pb)
                    fire_store(c - 1, pb)

                @pl.when(c >= NB)
                def _():
                    wait_store(b)
                wait_chg(b)
                fire_rows(b)

                @pl.when(c + 2 < NCHUNK)
                def _():
                    fire_cidx(c + 2, (b + 2) % NB)

                @pl.when(c + 1 < NCHUNK)
                def _():
                    nb_ = (b + 1) % NB
                    wait_cidx(nb_)
                    fire_chg(nb_)

    last_b = (NCHUNK - 1) % NB
    wait_rows(last_b)
    fire_store(NCHUNK - 1, last_b)
    for b in range(NB):
        wait_store(b)


def kernel(table, flat_charges, center_idx):
    return _two_level_gather(
        table,
        flat_charges.astype(jnp.int32),
        center_idx.astype(jnp.int32),
    )


# speedup vs baseline: 1.9729x; 1.0329x over previous
import functools

import jax
import jax.numpy as jnp
from jax import lax
from jax.experimental import pallas as pl
from jax.experimental.pallas import tpu as pltpu
from jax.experimental.pallas import tpu_sc as plsc

N_NUC = 10000
N_CENTER = 320000
MAX_CHARGE = 100
FEAT = 128

NC, NS = 2, 16
NW = NC * NS
PER_W = N_CENTER // NW
CHUNK = 200
NCHUNK = PER_W // CHUNK
NB = 4
NRING = (NCHUNK + NB - 1) // NB

_mesh = plsc.VectorSubcoreMesh(core_axis_name="c", subcore_axis_name="s")


@functools.partial(
    pl.kernel,
    out_type=jax.ShapeDtypeStruct((N_CENTER, FEAT), jnp.float32),
    mesh=_mesh,
    scratch_types=(
        [pltpu.VMEM_SHARED((N_NUC,), jnp.int32)]
        + [pltpu.VMEM_SHARED((MAX_CHARGE, FEAT), jnp.float32)]
        + [pltpu.VMEM((CHUNK,), jnp.int32) for _ in range(NB)]
        + [pltpu.VMEM((CHUNK,), jnp.int32) for _ in range(NB)]
        + [pltpu.VMEM((CHUNK, FEAT), jnp.float32) for _ in range(NB)]
        + [pltpu.SemaphoreType.DMA for _ in range(4 * NB)]
    ),
)
def _two_level_gather(table_hbm, charges_hbm, cidx_hbm, out_hbm, *refs):
    fc_s = refs[0]
    tab_s = refs[1]
    cidx_v = refs[2:2 + NB]
    chg_v = refs[2 + NB:2 + 2 * NB]
    rows_v = refs[2 + 2 * NB:2 + 3 * NB]
    sems = refs[2 + 3 * NB:]
    sem_i = sems[0:NB]
    sem_c = sems[NB:2 * NB]
    sem_r = sems[2 * NB:3 * NB]
    sem_s = sems[3 * NB:4 * NB]

    wid = lax.axis_index("s") * NC + lax.axis_index("c")
    base = wid * PER_W

    pltpu.async_copy(
        cidx_hbm.at[pl.ds(base, CHUNK)], cidx_v[0], sem_i[0])
    pltpu.async_copy(
        cidx_hbm.at[pl.ds(base + CHUNK, CHUNK)], cidx_v[1], sem_i[1])

    @pl.when(lax.axis_index("s") == 0)
    def _():
        pltpu.sync_copy(charges_hbm, fc_s)

    @pl.when(lax.axis_index("s") == 1)
    def _():
        pltpu.sync_copy(table_hbm, tab_s)
    plsc.subcore_barrier()

    def fire_cidx(c, b):
        pltpu.async_copy(
            cidx_hbm.at[pl.ds(base + c * CHUNK, CHUNK)], cidx_v[b], sem_i[b])

    def wait_cidx(b):
        pltpu.make_async_copy(
            cidx_hbm.at[pl.ds(0, CHUNK)], cidx_v[b], sem_i[b]).wait()

    def fire_chg(b):
        pltpu.async_copy(fc_s.at[cidx_v[b]], chg_v[b], sem_c[b])

    def wait_chg(b):
        pltpu.make_async_copy(fc_s.at[cidx_v[b]], chg_v[b], sem_c[b]).wait()

    def fire_rows(b):
        pltpu.async_copy(tab_s.at[chg_v[b]], rows_v[b], sem_r[b])

    def wait_rows(b):
        pltpu.make_async_copy(tab_s.at[chg_v[b]], rows_v[b], sem_r[b]).wait()

    def fire_store(c, b):
        pltpu.async_copy(
            rows_v[b], out_hbm.at[pl.ds(base + c * CHUNK, CHUNK)], sem_s[b])

    def wait_store(b):
        pltpu.make_async_copy(
            rows_v[b], out_hbm.at[pl.ds(0, CHUNK)], sem_s[b]).wait()

    wait_cidx(0)
    fire_chg(0)

    @pl.loop(0, NRING)
    def _(g):
        for b in range(NB):
            c = g * NB + b

            @pl.when(c < NCHUNK)
            def _():
                @pl.when(c >= NB)
                def _():
                    wait_store(b)
                wait_chg(b)
                fire_rows(b)

                @pl.when(c + 2 < NCHUNK)
                def _():
                    fire_cidx(c + 2, (b + 2) % NB)

                @pl.when(c + 1 < NCHUNK)
                def _():
                    nb_ = (b + 1) % NB
                    wait_cidx(nb_)
                    fire_chg(nb_)

                @pl.when(c >= 1)
                def _():
                    pb = (b - 1) % NB
                    wait_rows(pb)
                    fire_store(c - 1, pb)

    last_b = (NCHUNK - 1) % NB
    wait_rows(last_b)
    fire_store(NCHUNK - 1, last_b)
    for b in range(NB):
        wait_store(b)


def kernel(table, flat_charges, center_idx):
    return _two_level_gather(
        table,
        flat_charges.astype(jnp.int32),
        center_idx.astype(jnp.int32),
    )
